# baseline (speedup 1.0000x reference)
import functools

import jax
import jax.numpy as jnp
from jax import lax
from jax.experimental import pallas as pl
from jax.experimental.pallas import tpu as pltpu
from jax.experimental.pallas import tpu_sc as plsc

_K = 20
_NW = 32


def _make_sc_gather(D, B, dtype):
    assert B % (8 * _NW) == 0 and D % 8 == 0
    b_per_w = B // _NW
    ch = 128
    while ch * D * 4 > 130 * 1024 and ch > 8:
        ch //= 2
    while b_per_w % (2 * ch):
        ch //= 2
    n_ch = b_per_w // ch
    assert n_ch % 2 == 0
    mesh = plsc.VectorSubcoreMesh(core_axis_name="c", subcore_axis_name="s")

    @functools.partial(
        pl.kernel,
        mesh=mesh,
        compiler_params=pltpu.CompilerParams(use_tc_tiling_on_sc=False),
        out_type=jax.ShapeDtypeStruct((B, D), dtype),
        scratch_types=[
            pltpu.VMEM((b_per_w,), jnp.int32),
            pltpu.VMEM((ch, D), dtype),
            pltpu.VMEM((ch, D), dtype),
            pltpu.SemaphoreType.DMA,
            pltpu.SemaphoreType.DMA,
        ],
    )
    def k(table_hbm, idx_hbm, out_hbm, idx_v, rv0, rv1, s0, s1):
        wid = lax.axis_index("s") * 2 + lax.axis_index("c")
        base = pl.multiple_of(wid * b_per_w, 8)
        pltpu.sync_copy(idx_hbm.at[pl.ds(base, b_per_w)], idx_v)
        pltpu.async_copy(table_hbm.at[idx_v.at[pl.ds(0, ch)]], rv0, s0)

        def body(i, carry):
            o0 = pl.multiple_of(2 * i * ch, 8)
            o1 = pl.multiple_of((2 * i + 1) * ch, 8)
            pltpu.async_copy(table_hbm.at[idx_v.at[pl.ds(o1, ch)]], rv1, s1)
            pltpu.make_async_copy(
                table_hbm.at[idx_v.at[pl.ds(o0, ch)]], rv0, s0
            ).wait()
            pltpu.sync_copy(rv0, out_hbm.at[pl.ds(base + o0, ch)])

            @pl.when(i + 1 < n_ch // 2)
            def _():
                o2 = pl.multiple_of((2 * i + 2) * ch, 8)
                pltpu.async_copy(
                    table_hbm.at[idx_v.at[pl.ds(o2, ch)]], rv0, s0
                )

            pltpu.make_async_copy(
                table_hbm.at[idx_v.at[pl.ds(o1, ch)]], rv1, s1
            ).wait()
            pltpu.sync_copy(rv1, out_hbm.at[pl.ds(base + o1, ch)])
            return carry

        lax.fori_loop(0, n_ch // 2, body, 0)

    return k


def _sc_gather(table, idx):
    n, d = table.shape
    (b,) = idx.shape
    return _make_sc_gather(d, b, table.dtype)(table, idx)


def _global_idx(ni, vcount):
    bs = ni.shape[0]
    off = (jnp.arange(bs, dtype=jnp.int32) * vcount)[:, None, None]
    return (ni.astype(jnp.int32) + off).reshape(-1)


def _gather_nbr(t, idx):
    bs, v, c = t.shape
    _, r, n = idx.shape
    g = _sc_gather(t.reshape(bs * v, c), _global_idx(idx, v))
    return g.reshape(bs, r, n, c)


def _make_sc_plane_gather(N, B):
    b_per_w = B // _NW
    ch = 2048
    while b_per_w % ch or ch % 16:
        ch //= 2
    n_ch = b_per_w // ch
    mesh = plsc.VectorSubcoreMesh(core_axis_name="c", subcore_axis_name="s")

    @functools.partial(
        pl.kernel,
        mesh=mesh,
        compiler_params=pltpu.CompilerParams(
            use_tc_tiling_on_sc=False, needs_layout_passes=False
        ),
        out_type=[jax.ShapeDtypeStruct((B,), jnp.float32)] * 3,
        scratch_types=[
            pltpu.VMEM((N,), jnp.float32),
            pltpu.VMEM((N,), jnp.float32),
            pltpu.VMEM((N,), jnp.float32),
            pltpu.VMEM((ch,), jnp.int32),
            pltpu.VMEM((ch,), jnp.float32),
            pltpu.VMEM((ch,), jnp.float32),
            pltpu.VMEM((ch,), jnp.float32),
        ],
    )
    def k(xh, yh, zh, ih, oxh, oyh, ozh, xt, yt, zt, ib, ox, oy, oz):
        wid = lax.axis_index("s") * 2 + lax.axis_index("c")
        base = pl.multiple_of(wid * b_per_w, 8)
        pltpu.sync_copy(xh, xt)
        pltpu.sync_copy(yh, yt)
        pltpu.sync_copy(zh, zt)

        def body(c, carry):
            off = pl.multiple_of(base + c * ch, 8)
            pltpu.sync_copy(ih.at[pl.ds(off, ch)], ib)
            for j in range(ch // 16):
                iv = ib[pl.ds(j * 16, 16)]
                ox[pl.ds(j * 16, 16)] = plsc.load_gather(xt, [iv])
                oy[pl.ds(j * 16, 16)] = plsc.load_gather(yt, [iv])
                oz[pl.ds(j * 16, 16)] = plsc.load_gather(zt, [iv])
            pltpu.sync_copy(ox, oxh.at[pl.ds(off, ch)])
            pltpu.sync_copy(oy, oyh.at[pl.ds(off, ch)])
            pltpu.sync_copy(oz, ozh.at[pl.ds(off, ch)])
            return carry

        lax.fori_loop(0, n_ch, body, 0)

    return k


def _gather_vert(vt, idx):
    bs, _, v = vt.shape
    gi = _global_idx(idx, v)
    xp = vt[:, 0, :].reshape(bs * v)
    yp = vt[:, 1, :].reshape(bs * v)
    zp = vt[:, 2, :].reshape(bs * v)
    return _make_sc_plane_gather(bs * v, gi.shape[0])(xp, yp, zp, gi)


_CAP = 64


def _make_knn_prep(bs, V, RB):
    FW = min(256, V)
    nslots = V // FW

    def body(vt_ref, vtf_ref, d_ref, fold_ref, tau_ref):
        vrow = vt_ref[0]
        vall = vtf_ref[0]
        inner = lax.dot_general(vrow, vall, (((0,), (0,)), ((), ())))
        q = jnp.sum(vall * vall, axis=0)
        qr = jnp.sum(vrow * vrow, axis=0)
        d = -2.0 * inner + q[None, :] + qr[:, None]
        rblk = pl.program_id(1)
        rowg = rblk * RB + lax.broadcasted_iota(jnp.int32, (RB, V), 0)
        colg = lax.broadcasted_iota(jnp.int32, (RB, V), 1)
        d = jnp.where(rowg == colg, jnp.inf, d)
        d_ref[...] = d
        fold = jnp.full((RB, FW), jnp.inf, jnp.float32)
        for s in range(nslots):
            fold = jnp.minimum(fold, d[:, s * FW:(s + 1) * FW])
        fold_ref[...] = fold
        lane = lax.broadcasted_iota(jnp.int32, (RB, FW), 1)
        tau = None
        for _ in range(20):
            m = jnp.min(fold, axis=1)
            cand = jnp.where(fold == m[:, None], lane, 1 << 30)
            aml = jnp.min(cand, axis=1)
            fold = jnp.where(lane == aml[:, None], jnp.inf, fold)
            tau = m
        tau_ref[0] = jnp.broadcast_to(tau[:, None], (RB, 16))

    nrb = V // RB
    return pl.pallas_call(
        body,
        grid=(bs, nrb),
        in_specs=[
            pl.BlockSpec((1, 3, RB), lambda b, r: (b, 0, r)),
            pl.BlockSpec((1, 3, V), lambda b, r: (b, 0, 0)),
        ],
        out_specs=[
            pl.BlockSpec((RB, V), lambda b, r: (b * nrb + r, 0)),
            pl.BlockSpec((RB, FW), lambda b, r: (b * nrb + r, 0)),
            pl.BlockSpec((1, RB, 16), lambda b, r: (b * nrb + r, 0, 0)),
        ],
        out_shape=[
            jax.ShapeDtypeStruct((bs * V, V), jnp.float32),
            jax.ShapeDtypeStruct((bs * V, FW), jnp.float32),
            jax.ShapeDtypeStruct((bs * nrb, RB, 16), jnp.float32),
        ],
    )


def _make_sc_filter(R, V):
    rpt = R // _NW
    assert rpt % 16 == 0
    FW = min(256, V)
    mesh = plsc.VectorSubcoreMesh(core_axis_name="c", subcore_axis_name="s")
    inf16 = float('inf')

    @functools.partial(
        pl.kernel,
        mesh=mesh,
        compiler_params=pltpu.CompilerParams(
            use_tc_tiling_on_sc=False, needs_layout_passes=False
        ),
        out_type=[
            jax.ShapeDtypeStruct((R * _CAP,), jnp.float32),
            jax.ShapeDtypeStruct((R * _CAP,), jnp.int32),
        ],
        scratch_types=[
            pltpu.VMEM((8, V), jnp.float32),
            pltpu.VMEM((8, V), jnp.float32),
            pltpu.VMEM((8, FW), jnp.float32),
            pltpu.VMEM((8, FW), jnp.float32),
            pltpu.VMEM((rpt * 16,), jnp.float32),
            pltpu.VMEM((16 * _CAP,), jnp.float32),
            pltpu.VMEM((16 * _CAP,), jnp.int32),
            pltpu.VMEM((48,), jnp.int32),
            pltpu.SemaphoreType.DMA,
            pltpu.SemaphoreType.DMA,
            pltpu.SemaphoreType.DMA,
            pltpu.SemaphoreType.DMA,
        ],
    )
    def k(d_hbm, f_hbm, tau_hbm, val_hbm, idx_hbm, db0, db1, fb0, fb1,
          taub, vbuf, ibuf, hitb, s0, s1, s2, s3):
        wid = lax.axis_index("s") * 2 + lax.axis_index("c")
        base = pl.multiple_of(wid * rpt, 8)
        pltpu.sync_copy(tau_hbm.at[pl.ds(base * 16, rpt * 16)], taub)
        npos = V // FW

        def prefill():
            big = jnp.full((16,), jnp.inf, jnp.float32)
            for j in range(_CAP):
                vbuf[pl.ds(j * 16, 16)] = big

        def process(dref, fref, j, r):
            gi = lax.rem(r, 16)
            li = lax.iota(jnp.int32, 16)
            tauv = taub[pl.ds(pl.multiple_of(r * 16, 8), 16)]
            gb = jnp.full((16,), gi * _CAP, jnp.int32)
            jsplat = jnp.full((16,), j, jnp.int32)
            zero = jnp.zeros((16,), jnp.int32)
            for h in range(3):
                hitb[pl.ds(h * 16, 16)] = zero
            def fscan(fc, nh):
                fv = fref[j, pl.ds(pl.multiple_of(fc * 16, 8), 16)]
                fm = fv <= tauv
                cum = plsc.cumsum(jnp.where(fm, 1, 0))
                hpos = jnp.clip(nh + cum - 1, 0, 47)
                plsc.store_scatter(hitb, [hpos], li + fc * 16, mask=fm)
                return nh + plsc.all_reduce_population_count(fm)

            nh = lax.fori_loop(0, FW // 16, fscan, jnp.zeros((16,), jnp.int32))
            n = jnp.zeros((16,), jnp.int32)
            for h in range(3):
                hid = hitb[pl.ds(h * 16, 16)]
                valid = (li + h * 16) < nh
                for p in range(npos):
                    addr = hid + FW * p
                    lg = plsc.load_gather(dref, [jsplat, addr])
                    m = jnp.logical_and(lg <= tauv, valid)
                    cum = plsc.cumsum(jnp.where(m, 1, 0))
                    pos = jnp.clip(n + cum - 1, 0, _CAP - 1)
                    plsc.store_scatter(vbuf, [gb + pos], lg, mask=m)
                    plsc.store_scatter(ibuf, [gb + pos], addr, mask=m)
                    n = n + plsc.all_reduce_population_count(m)

        prefill()
        pltpu.async_copy(d_hbm.at[pl.ds(base, 8)], db0, s0)
        pltpu.async_copy(f_hbm.at[pl.ds(base, 8)], fb0, s2)

        def outer(i, carry):
            r0 = 16 * i
            pltpu.async_copy(d_hbm.at[pl.ds(base + r0 + 8, 8)], db1, s1)
            pltpu.async_copy(f_hbm.at[pl.ds(base + r0 + 8, 8)], fb1, s3)
            pltpu.make_async_copy(d_hbm.at[pl.ds(base + r0, 8)], db0, s0).wait()
            pltpu.make_async_copy(f_hbm.at[pl.ds(base + r0, 8)], fb0, s2).wait()
            for j in range(8):
                process(db0, fb0, j, r0 + j)

            @pl.when(i + 1 < rpt // 16)
            def _():
                pltpu.async_copy(d_hbm.at[pl.ds(base + r0 + 16, 8)], db0, s0)
                pltpu.async_copy(f_hbm.at[pl.ds(base + r0 + 16, 8)], fb0, s2)

            pltpu.make_async_copy(
                d_hbm.at[pl.ds(base + r0 + 8, 8)], db1, s1
            ).wait()
            pltpu.make_async_copy(
                f_hbm.at[pl.ds(base + r0 + 8, 8)], fb1, s3
            ).wait()
            for j in range(8):
                process(db1, fb1, j, r0 + 8 + j)

            off = pl.multiple_of((base + r0) * _CAP, 8)
            pltpu.sync_copy(vbuf, val_hbm.at[pl.ds(off, 16 * _CAP)])
            pltpu.sync_copy(ibuf, idx_hbm.at[pl.ds(off, 16 * _CAP)])
            prefill()
            return carry

        lax.fori_loop(0, rpt // 16, outer, 0)

    return k


def _make_knn_extract(R, RB):
    def body(val_ref, idx_ref, out_ref):
        v = val_ref[...]
        ii = idx_ref[...]
        lane = lax.broadcasted_iota(jnp.int32, (RB, _CAP), 1)
        for kk in range(20):
            m = jnp.min(v, axis=1)
            cand = jnp.where(v == m[:, None], lane, 1 << 30)
            aml = jnp.min(cand, axis=1)
            oh = lane == aml[:, None]
            nik = jnp.sum(jnp.where(oh, ii, 0), axis=1)
            out_ref[:, kk] = nik
            v = jnp.where(oh, jnp.inf, v)

    return pl.pallas_call(
        body,
        grid=(R // RB,),
        in_specs=[
            pl.BlockSpec((RB, _CAP), lambda r: (r, 0)),
            pl.BlockSpec((RB, _CAP), lambda r: (r, 0)),
        ],
        out_specs=pl.BlockSpec((RB, 32), lambda r: (r, 0)),
        out_shape=jax.ShapeDtypeStruct((R, 32), jnp.int32),
    )


def _knn_pallas(v):
    bs, V, _ = v.shape
    R = bs * V
    vt = jnp.transpose(v, (0, 2, 1))
    d, fold, tau = _make_knn_prep(bs, V, min(256, V))(vt, vt)
    cval, cidx = _make_sc_filter(R, V)(d, fold, tau.reshape(R * 16))
    ni = _make_knn_extract(R, 512)(
        cval.reshape(R, _CAP), cidx.reshape(R, _CAP)
    )
    return ni.reshape(bs, V, 32)[:, :, :20]


def _normalize(x, axis):
    norm = jnp.linalg.norm(x, axis=axis, keepdims=True)
    return x / jnp.maximum(norm, 1e-12)


def _knn(v, k):
    del k
    return _knn_pallas(v)


def _ndn_delta(v, ni):
    bs, vn, _ = v.shape
    vt = jnp.transpose(v, (0, 2, 1))
    gx, gy, gz = _gather_vert(vt, ni)
    R = bs * vn
    dx = gx.reshape(R, 20) - v[:, :, 0].reshape(R, 1)
    dy = gy.reshape(R, 20) - v[:, :, 1].reshape(R, 1)
    dz = gz.reshape(R, 20) - v[:, :, 2].reshape(R, 1)
    return dx, dy, dz


def _conv_rb(C):
    return {32: 128, 64: 128, 128: 64, 256: 32, 1024: 16}[C]


def _make_conv(R, C, have_feat, final_relu):
    RB = _conv_rb(C)
    assert R % RB == 0

    def body(*refs):
        if have_feat:
            dx_ref, dy_ref, dz_ref, dirs_ref, sg_ref, foc_ref, o_ref = refs
        else:
            dx_ref, dy_ref, dz_ref, dirs_ref, o_ref = refs
        dx = dx_ref[...]
        dy = dy_ref[...]
        dz = dz_ref[...]
        n2 = dx * dx + dy * dy + dz * dz
        inv = 1.0 / jnp.maximum(jnp.sqrt(n2), 1e-12)
        dx = dx * inv
        dy = dy * inv
        dz = dz * inv
        w = dirs_ref[...]
        wn = w / jnp.maximum(
            jnp.sqrt(jnp.sum(w * w, axis=0, keepdims=True)), 1e-12
        )
        w0 = wn[0].reshape(1, C)
        w1 = wn[1].reshape(1, C)
        w2 = wn[2].reshape(1, C)
        acc = None
        for j in range(20):
            theta = (
                dx[:, j:j + 1] * w0
                + dy[:, j:j + 1] * w1
                + dz[:, j:j + 1] * w2
            )
            theta = jnp.maximum(theta, 0.0)
            a = theta * sg_ref[j] if have_feat else theta
            acc = a if acc is None else jnp.maximum(acc, a)
        if have_feat:
            acc = acc + foc_ref[...]
        if final_relu:
            acc = jnp.maximum(acc, 0.0)
        o_ref[...] = acc

    in_specs = [
        pl.BlockSpec((RB, 20), lambda r: (r, 0)),
        pl.BlockSpec((RB, 20), lambda r: (r, 0)),
        pl.BlockSpec((RB, 20), lambda r: (r, 0)),
        pl.BlockSpec((3, C), lambda r: (0, 0)),
    ]
    if have_feat:
        in_specs += [
            pl.BlockSpec((20, RB, C), lambda r: (0, r, 0)),
            pl.BlockSpec((RB, C), lambda r: (r, 0)),
        ]
    return pl.pallas_call(
        body,
        grid=(R // RB,),
        in_specs=in_specs,
        out_specs=pl.BlockSpec((RB, C), lambda r: (r, 0)),
        out_shape=jax.ShapeDtypeStruct((R, C), jnp.float32),
    )


def _make_matmul(R, Cin, Cout):
    RB = 512 if R >= 512 else R

    def body(fm_ref, w_ref, b_ref, oc_ref, os_ref):
        fo = (
            jnp.dot(fm_ref[...], w_ref[...], preferred_element_type=jnp.float32)
            + b_ref[...]
        )
        oc_ref[...] = fo[:, :Cout]
        os_ref[...] = fo[:, Cout:]

    return pl.pallas_call(
        body,
        grid=(R // RB,),
        in_specs=[
            pl.BlockSpec((RB, Cin), lambda r: (r, 0)),
            pl.BlockSpec((Cin, 2 * Cout), lambda r: (0, 0)),
            pl.BlockSpec((1, 2 * Cout), lambda r: (0, 0)),
        ],
        out_specs=[
            pl.BlockSpec((RB, Cout), lambda r: (r, 0)),
            pl.BlockSpec((RB, Cout), lambda r: (r, 0)),
        ],
        out_shape=[
            jax.ShapeDtypeStruct((R, Cout), jnp.float32),
            jax.ShapeDtypeStruct((R, Cout), jnp.float32),
        ],
    )


def _make_pool_max(R, C):
    RB = _conv_rb(C)
    assert R % RB == 0

    def body(nf_ref, o_ref):
        acc = nf_ref[0]
        for j in range(1, 20):
            acc = jnp.maximum(acc, nf_ref[j])
        o_ref[...] = acc

    return pl.pallas_call(
        body,
        grid=(R // RB,),
        in_specs=[pl.BlockSpec((20, RB, C), lambda r: (0, r, 0))],
        out_specs=pl.BlockSpec((RB, C), lambda r: (r, 0)),
        out_shape=jax.ShapeDtypeStruct((R, C), jnp.float32),
    )


def _gather_nbr_t(t, idx):
    bs, v, c = t.shape
    _, r, n = idx.shape
    gi = _global_idx(idx, v).reshape(bs * r, n)
    gi = jnp.transpose(gi, (1, 0)).reshape(n * bs * r)
    g = _sc_gather(t.reshape(bs * v, c), gi)
    return g.reshape(n, bs * r, c)


def _conv_surface(dxyz, dirs):
    R = dxyz[0].shape[0]
    return _make_conv(R, dirs.shape[1], False, True)(*dxyz, dirs)


def _conv_layer(ni, dxyz, fm, w, b, dirs, oc, final_relu=True):
    bs, V, cin = fm.shape
    R = bs * V
    foc, fos = _make_matmul(R, cin, oc)(
        fm.reshape(R, cin), w, b.reshape(1, 2 * oc)
    )
    sg = _gather_nbr_t(fos.reshape(bs, V, oc), ni)
    out = _make_conv(R, oc, True, final_relu)(*dxyz, dirs, sg, foc)
    return out.reshape(bs, V, oc)


def _pool(v, fm, ni, rate):
    bs, vn, _ = v.shape
    c = fm.shape[2]
    samp = jnp.arange(vn // rate) * rate
    nf = _gather_nbr_t(fm, ni[:, samp, :])
    r2 = bs * (vn // rate)
    pooled = _make_pool_max(r2, c)(nf)
    return v[:, samp, :], pooled.reshape(bs, vn // rate, c)


def kernel(vertices, dirs0, w1, b1, dirs1, w2, b2, dirs2, w3, b3, dirs3, w4, b4, dirs4):
    bs, _, vn, _ = vertices.shape
    v = vertices.reshape(bs, vn, 3)
    ni = _knn(v, _K)
    dxyz = _ndn_delta(v, ni)
    fm0 = _conv_surface(dxyz, dirs0)
    fm0 = fm0.reshape(bs, vn, 32)
    fm1 = _conv_layer(ni, dxyz, fm0, w1, b1, dirs1, 64)
    v, fm1 = _pool(v, fm1, ni, 4)
    ni = _knn(v, _K)
    dxyz = _ndn_delta(v, ni)
    fm2 = _conv_layer(ni, dxyz, fm1, w2, b2, dirs2, 128)
    fm3 = _conv_layer(ni, dxyz, fm2, w3, b3, dirs3, 256)
    v, fm3 = _pool(v, fm3, ni, 4)
    ni = _knn(v, _K)
    dxyz = _ndn_delta(v, ni)
    fm4 = _conv_layer(ni, dxyz, fm3, w4, b4, dirs4, 1024, final_relu=False)
    fm4 = jnp.transpose(fm4, (0, 2, 1))[..., None]
    return fm4


# speedup vs baseline: 1.0035x; 1.0035x over previous
import functools

import jax
import jax.numpy as jnp
from jax import lax
from jax.experimental import pallas as pl
from jax.experimental.pallas import tpu as pltpu
from jax.experimental.pallas import tpu_sc as plsc

_K = 20
_NW = 32


def _make_sc_gather(D, B, dtype):
    assert B % (8 * _NW) == 0 and D % 8 == 0
    b_per_w = B // _NW
    ch = 128
    while ch * D * 4 > 120 * 1024 and ch > 8:
        ch //= 2
    while b_per_w % (4 * ch):
        ch //= 2
    n_ch = b_per_w // ch
    NB = 4
    assert n_ch % NB == 0
    mesh = plsc.VectorSubcoreMesh(core_axis_name="c", subcore_axis_name="s")

    @functools.partial(
        pl.kernel,
        mesh=mesh,
        compiler_params=pltpu.CompilerParams(use_tc_tiling_on_sc=False),
        out_type=jax.ShapeDtypeStruct((B, D), dtype),
        scratch_types=[
            pltpu.VMEM((b_per_w,), jnp.int32),
        ]
        + [pltpu.VMEM((ch, D), dtype)] * 4
        + [pltpu.SemaphoreType.DMA] * 4,
    )
    def k(table_hbm, idx_hbm, out_hbm, idx_v, rv0, rv1, rv2, rv3,
          s0, s1, s2, s3):
        rvs = (rv0, rv1, rv2, rv3)
        sems = (s0, s1, s2, s3)
        wid = lax.axis_index("s") * 2 + lax.axis_index("c")
        base = pl.multiple_of(wid * b_per_w, 8)
        pltpu.sync_copy(idx_hbm.at[pl.ds(base, b_per_w)], idx_v)

        def start(c, b):
            off = pl.multiple_of(c * ch, 8)
            pltpu.async_copy(
                table_hbm.at[idx_v.at[pl.ds(off, ch)]], rvs[b], sems[b]
            )

        def drain(c, b):
            off = pl.multiple_of(c * ch, 8)
            pltpu.make_async_copy(
                table_hbm.at[idx_v.at[pl.ds(off, ch)]], rvs[b], sems[b]
            ).wait()
            pltpu.sync_copy(rvs[b], out_hbm.at[pl.ds(base + off, ch)])

        for b in range(NB):
            start(b, b)

        def body(i, carry):
            c0 = NB * i
            for b in range(NB):
                drain(c0 + b, b)

                @pl.when(i + 1 < n_ch // NB)
                def _():
                    start(c0 + NB + b, b)

            return carry

        lax.fori_loop(0, n_ch // NB, body, 0)

    return k


def _sc_gather(table, idx):
    n, d = table.shape
    (b,) = idx.shape
    return _make_sc_gather(d, b, table.dtype)(table, idx)


def _global_idx(ni, vcount):
    bs = ni.shape[0]
    off = (jnp.arange(bs, dtype=jnp.int32) * vcount)[:, None, None]
    return (ni.astype(jnp.int32) + off).reshape(-1)


def _gather_nbr(t, idx):
    bs, v, c = t.shape
    _, r, n = idx.shape
    g = _sc_gather(t.reshape(bs * v, c), _global_idx(idx, v))
    return g.reshape(bs, r, n, c)


def _make_sc_plane_gather(N, B):
    b_per_w = B // _NW
    ch = 2048
    while b_per_w % ch or ch % 16:
        ch //= 2
    n_ch = b_per_w // ch
    mesh = plsc.VectorSubcoreMesh(core_axis_name="c", subcore_axis_name="s")

    @functools.partial(
        pl.kernel,
        mesh=mesh,
        compiler_params=pltpu.CompilerParams(
            use_tc_tiling_on_sc=False, needs_layout_passes=False
        ),
        out_type=[jax.ShapeDtypeStruct((B,), jnp.float32)] * 3,
        scratch_types=[
            pltpu.VMEM((N,), jnp.float32),
            pltpu.VMEM((N,), jnp.float32),
            pltpu.VMEM((N,), jnp.float32),
            pltpu.VMEM((ch,), jnp.int32),
            pltpu.VMEM((ch,), jnp.float32),
            pltpu.VMEM((ch,), jnp.float32),
            pltpu.VMEM((ch,), jnp.float32),
        ],
    )
    def k(xh, yh, zh, ih, oxh, oyh, ozh, xt, yt, zt, ib, ox, oy, oz):
        wid = lax.axis_index("s") * 2 + lax.axis_index("c")
        base = pl.multiple_of(wid * b_per_w, 8)
        pltpu.sync_copy(xh, xt)
        pltpu.sync_copy(yh, yt)
        pltpu.sync_copy(zh, zt)

        def body(c, carry):
            off = pl.multiple_of(base + c * ch, 8)
            pltpu.sync_copy(ih.at[pl.ds(off, ch)], ib)
            for j in range(ch // 16):
                iv = ib[pl.ds(j * 16, 16)]
                ox[pl.ds(j * 16, 16)] = plsc.load_gather(xt, [iv])
                oy[pl.ds(j * 16, 16)] = plsc.load_gather(yt, [iv])
                oz[pl.ds(j * 16, 16)] = plsc.load_gather(zt, [iv])
            pltpu.sync_copy(ox, oxh.at[pl.ds(off, ch)])
            pltpu.sync_copy(oy, oyh.at[pl.ds(off, ch)])
            pltpu.sync_copy(oz, ozh.at[pl.ds(off, ch)])
            return carry

        lax.fori_loop(0, n_ch, body, 0)

    return k


def _gather_vert(vt, idx):
    bs, _, v = vt.shape
    gi = _global_idx(idx, v)
    xp = vt[:, 0, :].reshape(bs * v)
    yp = vt[:, 1, :].reshape(bs * v)
    zp = vt[:, 2, :].reshape(bs * v)
    return _make_sc_plane_gather(bs * v, gi.shape[0])(xp, yp, zp, gi)


_CAP = 64


def _make_knn_prep(bs, V, RB):
    FW = min(256, V)
    nslots = V // FW

    def body(vt_ref, vtf_ref, d_ref, fold_ref, tau_ref):
        vrow = vt_ref[0]
        vall = vtf_ref[0]
        inner = lax.dot_general(vrow, vall, (((0,), (0,)), ((), ())))
        q = jnp.sum(vall * vall, axis=0)
        qr = jnp.sum(vrow * vrow, axis=0)
        d = -2.0 * inner + q[None, :] + qr[:, None]
        rblk = pl.program_id(1)
        rowg = rblk * RB + lax.broadcasted_iota(jnp.int32, (RB, V), 0)
        colg = lax.broadcasted_iota(jnp.int32, (RB, V), 1)
        d = jnp.where(rowg == colg, jnp.inf, d)
        d_ref[...] = d
        fold = jnp.full((RB, FW), jnp.inf, jnp.float32)
        for s in range(nslots):
            fold = jnp.minimum(fold, d[:, s * FW:(s + 1) * FW])
        fold_ref[...] = fold
        lane = lax.broadcasted_iota(jnp.int32, (RB, FW), 1)
        tau = None
        for _ in range(20):
            m = jnp.min(fold, axis=1)
            cand = jnp.where(fold == m[:, None], lane, 1 << 30)
            aml = jnp.min(cand, axis=1)
            fold = jnp.where(lane == aml[:, None], jnp.inf, fold)
            tau = m
        tau_ref[0] = jnp.broadcast_to(tau[:, None], (RB, 16))

    nrb = V // RB
    return pl.pallas_call(
        body,
        grid=(bs, nrb),
        in_specs=[
            pl.BlockSpec((1, 3, RB), lambda b, r: (b, 0, r)),
            pl.BlockSpec((1, 3, V), lambda b, r: (b, 0, 0)),
        ],
        out_specs=[
            pl.BlockSpec((RB, V), lambda b, r: (b * nrb + r, 0)),
            pl.BlockSpec((RB, FW), lambda b, r: (b * nrb + r, 0)),
            pl.BlockSpec((1, RB, 16), lambda b, r: (b * nrb + r, 0, 0)),
        ],
        out_shape=[
            jax.ShapeDtypeStruct((bs * V, V), jnp.float32),
            jax.ShapeDtypeStruct((bs * V, FW), jnp.float32),
            jax.ShapeDtypeStruct((bs * nrb, RB, 16), jnp.float32),
        ],
    )


def _make_sc_filter(R, V):
    rpt = R // _NW
    assert rpt % 16 == 0
    FW = min(256, V)
    mesh = plsc.VectorSubcoreMesh(core_axis_name="c", subcore_axis_name="s")
    inf16 = float('inf')

    @functools.partial(
        pl.kernel,
        mesh=mesh,
        compiler_params=pltpu.CompilerParams(
            use_tc_tiling_on_sc=False, needs_layout_passes=False
        ),
        out_type=[
            jax.ShapeDtypeStruct((R * _CAP,), jnp.float32),
            jax.ShapeDtypeStruct((R * _CAP,), jnp.int32),
        ],
        scratch_types=[
            pltpu.VMEM((8, V), jnp.float32),
            pltpu.VMEM((8, V), jnp.float32),
            pltpu.VMEM((8, FW), jnp.float32),
            pltpu.VMEM((8, FW), jnp.float32),
            pltpu.VMEM((rpt * 16,), jnp.float32),
            pltpu.VMEM((16 * _CAP,), jnp.float32),
            pltpu.VMEM((16 * _CAP,), jnp.int32),
            pltpu.VMEM((48,), jnp.int32),
            pltpu.SemaphoreType.DMA,
            pltpu.SemaphoreType.DMA,
            pltpu.SemaphoreType.DMA,
            pltpu.SemaphoreType.DMA,
        ],
    )
    def k(d_hbm, f_hbm, tau_hbm, val_hbm, idx_hbm, db0, db1, fb0, fb1,
          taub, vbuf, ibuf, hitb, s0, s1, s2, s3):
        wid = lax.axis_index("s") * 2 + lax.axis_index("c")
        base = pl.multiple_of(wid * rpt, 8)
        pltpu.sync_copy(tau_hbm.at[pl.ds(base * 16, rpt * 16)], taub)
        npos = V // FW

        def prefill():
            big = jnp.full((16,), jnp.inf, jnp.float32)
            for j in range(_CAP):
                vbuf[pl.ds(j * 16, 16)] = big

        def process(dref, fref, j, r):
            gi = lax.rem(r, 16)
            li = lax.iota(jnp.int32, 16)
            tauv = taub[pl.ds(pl.multiple_of(r * 16, 8), 16)]
            gb = jnp.full((16,), gi * _CAP, jnp.int32)
            jsplat = jnp.full((16,), j, jnp.int32)
            zero = jnp.zeros((16,), jnp.int32)
            for h in range(3):
                hitb[pl.ds(h * 16, 16)] = zero
            def fscan(fc, nh):
                fv = fref[j, pl.ds(pl.multiple_of(fc * 16, 8), 16)]
                fm = fv <= tauv
                cum = plsc.cumsum(jnp.where(fm, 1, 0))
                hpos = jnp.clip(nh + cum - 1, 0, 47)
                plsc.store_scatter(hitb, [hpos], li + fc * 16, mask=fm)
                return nh + plsc.all_reduce_population_count(fm)

            nh = lax.fori_loop(0, FW // 16, fscan, jnp.zeros((16,), jnp.int32))
            n = jnp.zeros((16,), jnp.int32)
            for h in range(3):
                hid = hitb[pl.ds(h * 16, 16)]
                valid = (li + h * 16) < nh
                for p in range(npos):
                    addr = hid + FW * p
                    lg = plsc.load_gather(dref, [jsplat, addr])
                    m = jnp.logical_and(lg <= tauv, valid)
                    cum = plsc.cumsum(jnp.where(m, 1, 0))
                    pos = jnp.clip(n + cum - 1, 0, _CAP - 1)
                    plsc.store_scatter(vbuf, [gb + pos], lg, mask=m)
                    plsc.store_scatter(ibuf, [gb + pos], addr, mask=m)
                    n = n + plsc.all_reduce_population_count(m)

        prefill()
        pltpu.async_copy(d_hbm.at[pl.ds(base, 8)], db0, s0)
        pltpu.async_copy(f_hbm.at[pl.ds(base, 8)], fb0, s2)

        def outer(i, carry):
            r0 = 16 * i
            pltpu.async_copy(d_hbm.at[pl.ds(base + r0 + 8, 8)], db1, s1)
            pltpu.async_copy(f_hbm.at[pl.ds(base + r0 + 8, 8)], fb1, s3)
            pltpu.make_async_copy(d_hbm.at[pl.ds(base + r0, 8)], db0, s0).wait()
            pltpu.make_async_copy(f_hbm.at[pl.ds(base + r0, 8)], fb0, s2).wait()
            for j in range(8):
                process(db0, fb0, j, r0 + j)

            @pl.when(i + 1 < rpt // 16)
            def _():
                pltpu.async_copy(d_hbm.at[pl.ds(base + r0 + 16, 8)], db0, s0)
                pltpu.async_copy(f_hbm.at[pl.ds(base + r0 + 16, 8)], fb0, s2)

            pltpu.make_async_copy(
                d_hbm.at[pl.ds(base + r0 + 8, 8)], db1, s1
            ).wait()
            pltpu.make_async_copy(
                f_hbm.at[pl.ds(base + r0 + 8, 8)], fb1, s3
            ).wait()
            for j in range(8):
                process(db1, fb1, j, r0 + 8 + j)

            off = pl.multiple_of((base + r0) * _CAP, 8)
            pltpu.sync_copy(vbuf, val_hbm.at[pl.ds(off, 16 * _CAP)])
            pltpu.sync_copy(ibuf, idx_hbm.at[pl.ds(off, 16 * _CAP)])
            prefill()
            return carry

        lax.fori_loop(0, rpt // 16, outer, 0)

    return k


def _make_knn_extract(R, RB):
    def body(val_ref, idx_ref, out_ref):
        v = val_ref[...]
        ii = idx_ref[...]
        lane = lax.broadcasted_iota(jnp.int32, (RB, _CAP), 1)
        for kk in range(20):
            m = jnp.min(v, axis=1)
            cand = jnp.where(v == m[:, None], lane, 1 << 30)
            aml = jnp.min(cand, axis=1)
            oh = lane == aml[:, None]
            nik = jnp.sum(jnp.where(oh, ii, 0), axis=1)
            out_ref[:, kk] = nik
            v = jnp.where(oh, jnp.inf, v)

    return pl.pallas_call(
        body,
        grid=(R // RB,),
        in_specs=[
            pl.BlockSpec((RB, _CAP), lambda r: (r, 0)),
            pl.BlockSpec((RB, _CAP), lambda r: (r, 0)),
        ],
        out_specs=pl.BlockSpec((RB, 32), lambda r: (r, 0)),
        out_shape=jax.ShapeDtypeStruct((R, 32), jnp.int32),
    )


def _knn_pallas(v):
    bs, V, _ = v.shape
    R = bs * V
    vt = jnp.transpose(v, (0, 2, 1))
    d, fold, tau = _make_knn_prep(bs, V, min(256, V))(vt, vt)
    cval, cidx = _make_sc_filter(R, V)(d, fold, tau.reshape(R * 16))
    ni = _make_knn_extract(R, 512)(
        cval.reshape(R, _CAP), cidx.reshape(R, _CAP)
    )
    return ni.reshape(bs, V, 32)[:, :, :20]


def _normalize(x, axis):
    norm = jnp.linalg.norm(x, axis=axis, keepdims=True)
    return x / jnp.maximum(norm, 1e-12)


def _knn(v, k):
    del k
    return _knn_pallas(v)


def _ndn_delta(v, ni):
    bs, vn, _ = v.shape
    vt = jnp.transpose(v, (0, 2, 1))
    gx, gy, gz = _gather_vert(vt, ni)
    R = bs * vn
    dx = gx.reshape(R, 20) - v[:, :, 0].reshape(R, 1)
    dy = gy.reshape(R, 20) - v[:, :, 1].reshape(R, 1)
    dz = gz.reshape(R, 20) - v[:, :, 2].reshape(R, 1)
    return dx, dy, dz


def _conv_rb(C):
    return {32: 128, 64: 128, 128: 64, 256: 32, 1024: 16}[C]


def _make_conv(R, C, have_feat, final_relu):
    RB = _conv_rb(C)
    assert R % RB == 0

    def body(*refs):
        if have_feat:
            dx_ref, dy_ref, dz_ref, dirs_ref, sg_ref, foc_ref, o_ref = refs
        else:
            dx_ref, dy_ref, dz_ref, dirs_ref, o_ref = refs
        dx = dx_ref[...]
        dy = dy_ref[...]
        dz = dz_ref[...]
        n2 = dx * dx + dy * dy + dz * dz
        inv = 1.0 / jnp.maximum(jnp.sqrt(n2), 1e-12)
        dx = dx * inv
        dy = dy * inv
        dz = dz * inv
        w = dirs_ref[...]
        wn = w / jnp.maximum(
            jnp.sqrt(jnp.sum(w * w, axis=0, keepdims=True)), 1e-12
        )
        w0 = wn[0].reshape(1, C)
        w1 = wn[1].reshape(1, C)
        w2 = wn[2].reshape(1, C)
        acc = None
        for j in range(20):
            theta = (
                dx[:, j:j + 1] * w0
                + dy[:, j:j + 1] * w1
                + dz[:, j:j + 1] * w2
            )
            theta = jnp.maximum(theta, 0.0)
            a = theta * sg_ref[j] if have_feat else theta
            acc = a if acc is None else jnp.maximum(acc, a)
        if have_feat:
            acc = acc + foc_ref[...]
        if final_relu:
            acc = jnp.maximum(acc, 0.0)
        o_ref[...] = acc

    in_specs = [
        pl.BlockSpec((RB, 20), lambda r: (r, 0)),
        pl.BlockSpec((RB, 20), lambda r: (r, 0)),
        pl.BlockSpec((RB, 20), lambda r: (r, 0)),
        pl.BlockSpec((3, C), lambda r: (0, 0)),
    ]
    if have_feat:
        in_specs += [
            pl.BlockSpec((20, RB, C), lambda r: (0, r, 0)),
            pl.BlockSpec((RB, C), lambda r: (r, 0)),
        ]
    return pl.pallas_call(
        body,
        grid=(R // RB,),
        in_specs=in_specs,
        out_specs=pl.BlockSpec((RB, C), lambda r: (r, 0)),
        out_shape=jax.ShapeDtypeStruct((R, C), jnp.float32),
    )


def _make_matmul(R, Cin, Cout):
    RB = 512 if R >= 512 else R

    def body(fm_ref, w_ref, b_ref, oc_ref, os_ref):
        fo = (
            jnp.dot(fm_ref[...], w_ref[...], preferred_element_type=jnp.float32)
            + b_ref[...]
        )
        oc_ref[...] = fo[:, :Cout]
        os_ref[...] = fo[:, Cout:]

    return pl.pallas_call(
        body,
        grid=(R // RB,),
        in_specs=[
            pl.BlockSpec((RB, Cin), lambda r: (r, 0)),
            pl.BlockSpec((Cin, 2 * Cout), lambda r: (0, 0)),
            pl.BlockSpec((1, 2 * Cout), lambda r: (0, 0)),
        ],
        out_specs=[
            pl.BlockSpec((RB, Cout), lambda r: (r, 0)),
            pl.BlockSpec((RB, Cout), lambda r: (r, 0)),
        ],
        out_shape=[
            jax.ShapeDtypeStruct((R, Cout), jnp.float32),
            jax.ShapeDtypeStruct((R, Cout), jnp.float32),
        ],
    )


def _make_pool_max(R, C):
    RB = _conv_rb(C)
    assert R % RB == 0

    def body(nf_ref, o_ref):
        acc = nf_ref[0]
        for j in range(1, 20):
            acc = jnp.maximum(acc, nf_ref[j])
        o_ref[...] = acc

    return pl.pallas_call(
        body,
        grid=(R // RB,),
        in_specs=[pl.BlockSpec((20, RB, C), lambda r: (0, r, 0))],
        out_specs=pl.BlockSpec((RB, C), lambda r: (r, 0)),
        out_shape=jax.ShapeDtypeStruct((R, C), jnp.float32),
    )


def _gather_nbr_t(t, idx):
    bs, v, c = t.shape
    _, r, n = idx.shape
    gi = _global_idx(idx, v).reshape(bs * r, n)
    gi = jnp.transpose(gi, (1, 0)).reshape(n * bs * r)
    g = _sc_gather(t.reshape(bs * v, c), gi)
    return g.reshape(n, bs * r, c)


def _conv_surface(dxyz, dirs):
    R = dxyz[0].shape[0]
    return _make_conv(R, dirs.shape[1], False, True)(*dxyz, dirs)


def _conv_layer(ni, dxyz, fm, w, b, dirs, oc, final_relu=True):
    bs, V, cin = fm.shape
    R = bs * V
    foc, fos = _make_matmul(R, cin, oc)(
        fm.reshape(R, cin), w, b.reshape(1, 2 * oc)
    )
    sg = _gather_nbr_t(fos.reshape(bs, V, oc), ni)
    out = _make_conv(R, oc, True, final_relu)(*dxyz, dirs, sg, foc)
    return out.reshape(bs, V, oc)


def _pool(v, fm, ni, rate):
    bs, vn, _ = v.shape
    c = fm.shape[2]
    samp = jnp.arange(vn // rate) * rate
    nf = _gather_nbr_t(fm, ni[:, samp, :])
    r2 = bs * (vn // rate)
    pooled = _make_pool_max(r2, c)(nf)
    return v[:, samp, :], pooled.reshape(bs, vn // rate, c)


def kernel(vertices, dirs0, w1, b1, dirs1, w2, b2, dirs2, w3, b3, dirs3, w4, b4, dirs4):
    bs, _, vn, _ = vertices.shape
    v = vertices.reshape(bs, vn, 3)
    ni = _knn(v, _K)
    dxyz = _ndn_delta(v, ni)
    fm0 = _conv_surface(dxyz, dirs0)
    fm0 = fm0.reshape(bs, vn, 32)
    fm1 = _conv_layer(ni, dxyz, fm0, w1, b1, dirs1, 64)
    v, fm1 = _pool(v, fm1, ni, 4)
    ni = _knn(v, _K)
    dxyz = _ndn_delta(v, ni)
    fm2 = _conv_layer(ni, dxyz, fm1, w2, b2, dirs2, 128)
    fm3 = _conv_layer(ni, dxyz, fm2, w3, b3, dirs3, 256)
    v, fm3 = _pool(v, fm3, ni, 4)
    ni = _knn(v, _K)
    dxyz = _ndn_delta(v, ni)
    fm4 = _conv_layer(ni, dxyz, fm3, w4, b4, dirs4, 1024, final_relu=False)
    fm4 = jnp.transpose(fm4, (0, 2, 1))[..., None]
    return fm4
